# Initial kernel scaffold; baseline (speedup 1.0000x reference)
import jax
import jax.numpy as jnp
from jax.experimental import pallas as pl


def kernel(A_pre, A_qs, A_vs, A_rs, A_uq, S, Q, V, R, U, gat_W, gat_a_src, gat_a_dst, q_W, q_b, v_W, v_b, r_W, r_b, u_W, u_b):
    raise NotImplementedError("write your pallas kernel here")


# speedup vs baseline: 1.0859x; 1.0859x over previous
import jax
import jax.numpy as jnp
import numpy as np
from jax.experimental import pallas as pl
from jax.experimental.pallas import tpu as pltpu

N_S, N_Q, N_V, N_R, N_U = 2048, 8192, 4096, 4096, 8192
D, H, DH = 128, 4, 32
BN_SCALE = float(1.0 / np.sqrt(1.0 + 1e-5))
F32 = jnp.float32


def _gat_pre_body(x_ref, w_ref, asrc_ref, adst_ref, sel_ref, xw_ref, es_ref, edt_ref):
    xw = jnp.dot(x_ref[...], w_ref[...], preferred_element_type=F32)
    xw_ref[...] = xw
    sel = sel_ref[...]
    es_ref[...] = jnp.dot(xw * asrc_ref[...], sel, preferred_element_type=F32)
    td = xw * adst_ref[...]
    edt_ref[...] = jax.lax.dot_general(
        sel, td, (((0,), (1,)), ((), ())), preferred_element_type=F32)


def _gat_main_body(final):
    def body(a_ref, xw_ref, es_ref, edt_ref, s_ref, o_ref):
        a = a_ref[...]
        mask = a > 0
        xw = xw_ref[...]
        es = es_ref[...]
        edt = edt_ref[...]
        outs = []
        for h in range(H):
            z = es[:, h:h + 1] + edt[h:h + 1, :]
            e = jnp.where(z >= 0, z, 0.2 * z)
            e = jnp.where(mask, e, -1e9)
            m = jnp.max(e, axis=1, keepdims=True)
            p = jnp.exp(e - m)
            ssum = jnp.sum(p, axis=1, keepdims=True)
            alpha = p / ssum
            outs.append(jnp.dot(alpha, xw[:, h * DH:(h + 1) * DH],
                                preferred_element_type=F32))
        out = jnp.concatenate(outs, axis=1) * BN_SCALE
        if final:
            v = out + s_ref[...]
            out = jnp.where(v > 0, v, jnp.exp(jnp.minimum(v, 0.0)) - 1.0)
        o_ref[...] = out
    return body


def _gat_layer(X, A, W_flat, a_src_flat, a_dst_flat, sel, S0, final):
    xw, es, edt = pl.pallas_call(
        _gat_pre_body,
        out_shape=[jax.ShapeDtypeStruct((N_S, D), F32),
                   jax.ShapeDtypeStruct((N_S, H), F32),
                   jax.ShapeDtypeStruct((H, N_S), F32)],
    )(X, W_flat, a_src_flat, a_dst_flat, sel)

    BN = 256
    out = pl.pallas_call(
        _gat_main_body(final),
        grid=(N_S // BN,),
        in_specs=[
            pl.BlockSpec((BN, N_S), lambda i: (i, 0)),
            pl.BlockSpec((N_S, D), lambda i: (0, 0)),
            pl.BlockSpec((BN, H), lambda i: (i, 0)),
            pl.BlockSpec((H, N_S), lambda i: (0, 0)),
            pl.BlockSpec((BN, D), lambda i: (i, 0)),
        ],
        out_specs=pl.BlockSpec((BN, D), lambda i: (i, 0)),
        out_shape=jax.ShapeDtypeStruct((N_S, D), F32),
    )(A, xw, es, edt, S0)
    return out


def _bip_fwd_body(a_ref, x_ref, w_ref, b_ref, o_ref, acc_ref, rs_ref):
    k = pl.program_id(1)
    nk = pl.num_programs(1)
    a = a_ref[...]
    part = jnp.dot(a, x_ref[...], preferred_element_type=F32)
    rpart = jnp.sum(a, axis=1, keepdims=True)

    @pl.when(k == 0)
    def _():
        acc_ref[...] = part
        rs_ref[...] = rpart

    @pl.when(k != 0)
    def _():
        acc_ref[...] += part
        rs_ref[...] += rpart

    @pl.when(k == nk - 1)
    def _():
        z = acc_ref[...] / jnp.clip(rs_ref[...], 1e-8, None)
        hh = jnp.dot(z, w_ref[...], preferred_element_type=F32) + b_ref[...]
        o_ref[...] = jnp.maximum(hh * BN_SCALE, 0.0)


def _bip_fwd(A, X, W, b, bt, bk):
    T, S = A.shape
    return pl.pallas_call(
        _bip_fwd_body,
        grid=(T // bt, S // bk),
        in_specs=[
            pl.BlockSpec((bt, bk), lambda i, k: (i, k)),
            pl.BlockSpec((bk, D), lambda i, k: (k, 0)),
            pl.BlockSpec((D, D), lambda i, k: (0, 0)),
            pl.BlockSpec((1, D), lambda i, k: (0, 0)),
        ],
        out_specs=pl.BlockSpec((bt, D), lambda i, k: (i, 0)),
        out_shape=jax.ShapeDtypeStruct((T, D), F32),
        scratch_shapes=[pltpu.VMEM((bt, D), F32), pltpu.VMEM((bt, 1), F32)],
    )(A, X, W, b)


def _bip_bwd_body(a_ref, y_ref, w_ref, b_ref, o_ref, acc_ref, cs_ref):
    t = pl.program_id(1)
    nt = pl.num_programs(1)
    a = a_ref[...]
    part = jax.lax.dot_general(
        a, y_ref[...], (((0,), (0,)), ((), ())), preferred_element_type=F32)
    cpart = jnp.sum(a, axis=0, keepdims=True)

    @pl.when(t == 0)
    def _():
        acc_ref[...] = part
        cs_ref[...] = cpart

    @pl.when(t != 0)
    def _():
        acc_ref[...] += part
        cs_ref[...] += cpart

    @pl.when(t == nt - 1)
    def _():
        cs_col = jnp.transpose(cs_ref[...])
        z = acc_ref[...] / jnp.clip(cs_col, 1e-8, None)
        hh = jnp.dot(z, w_ref[...], preferred_element_type=F32) + b_ref[...]
        o_ref[...] = jnp.maximum(hh * BN_SCALE, 0.0)


def _bip_bwd(A, Y, W, b, bs, bkt):
    T, S = A.shape
    return pl.pallas_call(
        _bip_bwd_body,
        grid=(S // bs, T // bkt),
        in_specs=[
            pl.BlockSpec((bkt, bs), lambda i, t: (t, i)),
            pl.BlockSpec((bkt, D), lambda i, t: (t, 0)),
            pl.BlockSpec((D, D), lambda i, t: (0, 0)),
            pl.BlockSpec((1, D), lambda i, t: (0, 0)),
        ],
        out_specs=pl.BlockSpec((bs, D), lambda i, t: (i, 0)),
        out_shape=jax.ShapeDtypeStruct((S, D), F32),
        scratch_shapes=[pltpu.VMEM((bs, D), F32), pltpu.VMEM((1, bs), F32)],
    )(A, Y, W, b)


def _bip_stack(h_src, A, Ws, bs_, bt_f, bk_f, bs_b, bkt_b):
    h1 = _bip_fwd(A, h_src, Ws[0], bs_[0], bt_f, bk_f)
    h2 = _bip_bwd(A, h1, Ws[1], bs_[1], bs_b, bkt_b)
    h3 = _bip_fwd(A, h2, Ws[2], bs_[2], bt_f, bk_f)
    return h3


def kernel(A_pre, A_qs, A_vs, A_rs, A_uq, S, Q, V, R, U,
           gat_W, gat_a_src, gat_a_dst,
           q_W, q_b, v_W, v_b, r_W, r_b, u_W, u_b):
    sel = jnp.repeat(jnp.eye(H, dtype=F32), DH, axis=0)

    x = S
    for i in range(3):
        x = _gat_layer(
            x, A_pre,
            gat_W[i].reshape(D, H * DH),
            gat_a_src[i].reshape(1, H * DH),
            gat_a_dst[i].reshape(1, H * DH),
            sel, S, final=(i == 2))
    h_s = x

    def _b(b):
        return [b[l].reshape(1, D) for l in range(3)]

    h_qa = _bip_stack(h_s, A_qs, q_W, _b(q_b), 512, 2048, 512, 2048)
    h_v = _bip_stack(h_s, A_vs, v_W, _b(v_b), 512, 2048, 512, 2048)
    h_r = _bip_stack(h_s, A_rs, r_W, _b(r_b), 512, 2048, 512, 2048)
    h_u = _bip_stack(h_qa, A_uq, u_W, _b(u_b), 512, 2048, 512, 2048)

    return jnp.concatenate([h_qa, h_v, h_r, h_u], axis=0)


# speedup vs baseline: 1.2377x; 1.1398x over previous
import jax
import jax.numpy as jnp
import numpy as np
from jax.experimental import pallas as pl
from jax.experimental.pallas import tpu as pltpu

N_S, N_Q, N_V, N_R, N_U = 2048, 8192, 4096, 4096, 8192
D, H, DH = 128, 4, 32
BN_SCALE = float(1.0 / np.sqrt(1.0 + 1e-5))
F32 = jnp.float32


def _gat_pre_body(x_ref, w_ref, asrc_ref, adst_ref, sel_ref, xw_ref, es_ref, edt_ref):
    xw = jnp.dot(x_ref[...], w_ref[...], preferred_element_type=F32)
    xw_ref[...] = xw
    sel = sel_ref[...]
    es_ref[...] = jnp.dot(xw * asrc_ref[...], sel, preferred_element_type=F32)
    td = xw * adst_ref[...]
    edt_ref[...] = jax.lax.dot_general(
        sel, td, (((0,), (1,)), ((), ())), preferred_element_type=F32)


def _gat_main_body(final):
    def body(a_ref, xw_ref, es_ref, edt_ref, s_ref, o_ref):
        a = a_ref[...]
        mask = a > 0
        xw = xw_ref[...]
        es = es_ref[...]
        edt = edt_ref[...]
        outs = []
        for h in range(H):
            z = es[:, h:h + 1] + edt[h:h + 1, :]
            e = jnp.where(z >= 0, z, 0.2 * z)
            e = jnp.where(mask, e, -1e9)
            m = jnp.max(e, axis=1, keepdims=True)
            p = jnp.exp(e - m)
            ssum = jnp.sum(p, axis=1, keepdims=True)
            alpha = p / ssum
            outs.append(jnp.dot(alpha, xw[:, h * DH:(h + 1) * DH],
                                preferred_element_type=F32))
        out = jnp.concatenate(outs, axis=1) * BN_SCALE
        if final:
            v = out + s_ref[...]
            out = jnp.where(v > 0, v, jnp.exp(jnp.minimum(v, 0.0)) - 1.0)
        o_ref[...] = out
    return body


def _gat_layer(X, A, W_flat, a_src_flat, a_dst_flat, sel, S0, final):
    xw, es, edt = pl.pallas_call(
        _gat_pre_body,
        out_shape=[jax.ShapeDtypeStruct((N_S, D), F32),
                   jax.ShapeDtypeStruct((N_S, H), F32),
                   jax.ShapeDtypeStruct((H, N_S), F32)],
    )(X, W_flat, a_src_flat, a_dst_flat, sel)

    BN = 256
    out = pl.pallas_call(
        _gat_main_body(final),
        grid=(N_S // BN,),
        in_specs=[
            pl.BlockSpec((BN, N_S), lambda i: (i, 0)),
            pl.BlockSpec((N_S, D), lambda i: (0, 0)),
            pl.BlockSpec((BN, H), lambda i: (i, 0)),
            pl.BlockSpec((H, N_S), lambda i: (0, 0)),
            pl.BlockSpec((BN, D), lambda i: (i, 0)),
        ],
        out_specs=pl.BlockSpec((BN, D), lambda i: (i, 0)),
        out_shape=jax.ShapeDtypeStruct((N_S, D), F32),
    )(A, xw, es, edt, S0)
    return out


def _bip_fwd_body(a_ref, x_ref, w_ref, b_ref, o_ref, acc_ref, rs_ref):
    k = pl.program_id(1)
    nk = pl.num_programs(1)
    a = a_ref[...]
    part = jnp.dot(a, x_ref[...], preferred_element_type=F32)
    rpart = jnp.sum(a, axis=1, keepdims=True)

    @pl.when(k == 0)
    def _():
        acc_ref[...] = part
        rs_ref[...] = rpart

    @pl.when(k != 0)
    def _():
        acc_ref[...] += part
        rs_ref[...] += rpart

    @pl.when(k == nk - 1)
    def _():
        z = acc_ref[...] / jnp.clip(rs_ref[...], 1e-8, None)
        hh = jnp.dot(z, w_ref[...], preferred_element_type=F32) + b_ref[...]
        o_ref[...] = jnp.maximum(hh * BN_SCALE, 0.0)


def _bip_fwd(A, X, W, b, bt, bk):
    T, S = A.shape
    return pl.pallas_call(
        _bip_fwd_body,
        grid=(T // bt, S // bk),
        in_specs=[
            pl.BlockSpec((bt, bk), lambda i, k: (i, k)),
            pl.BlockSpec((bk, D), lambda i, k: (k, 0)),
            pl.BlockSpec((D, D), lambda i, k: (0, 0)),
            pl.BlockSpec((1, D), lambda i, k: (0, 0)),
        ],
        out_specs=pl.BlockSpec((bt, D), lambda i, k: (i, 0)),
        out_shape=jax.ShapeDtypeStruct((T, D), F32),
        scratch_shapes=[pltpu.VMEM((bt, D), F32), pltpu.VMEM((bt, 1), F32)],
    )(A, X, W, b)


def _bip_fused12_body(a_ref, x_ref, w0_ref, b0_ref, w1_ref, b1_ref,
                      o_ref, acc_ref, cs_ref):
    t = pl.program_id(0)
    nt = pl.num_programs(0)
    a = a_ref[...]
    rs = jnp.clip(jnp.sum(a, axis=1, keepdims=True), 1e-8, None)
    z = jnp.dot(a, x_ref[...], preferred_element_type=F32) / rs
    y = jnp.maximum((jnp.dot(z, w0_ref[...], preferred_element_type=F32)
                     + b0_ref[...]) * BN_SCALE, 0.0)
    part2 = jax.lax.dot_general(
        a, y, (((0,), (0,)), ((), ())), preferred_element_type=F32)
    cpart = jnp.sum(a, axis=0, keepdims=True)

    @pl.when(t == 0)
    def _():
        acc_ref[...] = part2
        cs_ref[...] = cpart

    @pl.when(t != 0)
    def _():
        acc_ref[...] += part2
        cs_ref[...] += cpart

    @pl.when(t == nt - 1)
    def _():
        cs_col = jnp.transpose(cs_ref[...])
        z2 = acc_ref[...] / jnp.clip(cs_col, 1e-8, None)
        hh = jnp.dot(z2, w1_ref[...], preferred_element_type=F32) + b1_ref[...]
        o_ref[...] = jnp.maximum(hh * BN_SCALE, 0.0)


def _bip_fused12(A, X, W0, b0, W1, b1, bt):
    T, S = A.shape
    return pl.pallas_call(
        _bip_fused12_body,
        grid=(T // bt,),
        in_specs=[
            pl.BlockSpec((bt, S), lambda t: (t, 0)),
            pl.BlockSpec((S, D), lambda t: (0, 0)),
            pl.BlockSpec((D, D), lambda t: (0, 0)),
            pl.BlockSpec((1, D), lambda t: (0, 0)),
            pl.BlockSpec((D, D), lambda t: (0, 0)),
            pl.BlockSpec((1, D), lambda t: (0, 0)),
        ],
        out_specs=pl.BlockSpec((S, D), lambda t: (0, 0)),
        out_shape=jax.ShapeDtypeStruct((S, D), F32),
        scratch_shapes=[pltpu.VMEM((S, D), F32), pltpu.VMEM((1, S), F32)],
    )(A, X, W0, b0, W1, b1)


def _bip_bwd_body(a_ref, y_ref, w_ref, b_ref, o_ref, acc_ref, cs_ref):
    t = pl.program_id(1)
    nt = pl.num_programs(1)
    a = a_ref[...]
    part = jax.lax.dot_general(
        a, y_ref[...], (((0,), (0,)), ((), ())), preferred_element_type=F32)
    cpart = jnp.sum(a, axis=0, keepdims=True)

    @pl.when(t == 0)
    def _():
        acc_ref[...] = part
        cs_ref[...] = cpart

    @pl.when(t != 0)
    def _():
        acc_ref[...] += part
        cs_ref[...] += cpart

    @pl.when(t == nt - 1)
    def _():
        cs_col = jnp.transpose(cs_ref[...])
        z = acc_ref[...] / jnp.clip(cs_col, 1e-8, None)
        hh = jnp.dot(z, w_ref[...], preferred_element_type=F32) + b_ref[...]
        o_ref[...] = jnp.maximum(hh * BN_SCALE, 0.0)


def _bip_bwd(A, Y, W, b, bs, bkt):
    T, S = A.shape
    return pl.pallas_call(
        _bip_bwd_body,
        grid=(S // bs, T // bkt),
        in_specs=[
            pl.BlockSpec((bkt, bs), lambda i, t: (t, i)),
            pl.BlockSpec((bkt, D), lambda i, t: (t, 0)),
            pl.BlockSpec((D, D), lambda i, t: (0, 0)),
            pl.BlockSpec((1, D), lambda i, t: (0, 0)),
        ],
        out_specs=pl.BlockSpec((bs, D), lambda i, t: (i, 0)),
        out_shape=jax.ShapeDtypeStruct((S, D), F32),
        scratch_shapes=[pltpu.VMEM((bs, D), F32), pltpu.VMEM((1, bs), F32)],
    )(A, Y, W, b)


def _bip_stack(h_src, A, Ws, bs_, bt_f, bk_f, bt_12):
    h2 = _bip_fused12(A, h_src, Ws[0], bs_[0], Ws[1], bs_[1], bt_12)
    h3 = _bip_fwd(A, h2, Ws[2], bs_[2], bt_f, bk_f)
    return h3


def kernel(A_pre, A_qs, A_vs, A_rs, A_uq, S, Q, V, R, U,
           gat_W, gat_a_src, gat_a_dst,
           q_W, q_b, v_W, v_b, r_W, r_b, u_W, u_b):
    sel = jnp.repeat(jnp.eye(H, dtype=F32), DH, axis=0)

    x = S
    for i in range(3):
        x = _gat_layer(
            x, A_pre,
            gat_W[i].reshape(D, H * DH),
            gat_a_src[i].reshape(1, H * DH),
            gat_a_dst[i].reshape(1, H * DH),
            sel, S, final=(i == 2))
    h_s = x

    def _b(b):
        return [b[l].reshape(1, D) for l in range(3)]

    h_qa = _bip_stack(h_s, A_qs, q_W, _b(q_b), 512, 2048, 512)
    h_v = _bip_stack(h_s, A_vs, v_W, _b(v_b), 512, 2048, 512)
    h_r = _bip_stack(h_s, A_rs, r_W, _b(r_b), 512, 2048, 512)
    h_u = _bip_stack(h_qa, A_uq, u_W, _b(u_b), 512, 2048, 256)

    return jnp.concatenate([h_qa, h_v, h_r, h_u], axis=0)


# speedup vs baseline: 1.4079x; 1.1375x over previous
import jax
import jax.numpy as jnp
import numpy as np
from jax.experimental import pallas as pl
from jax.experimental.pallas import tpu as pltpu

N_S, N_Q, N_V, N_R, N_U = 2048, 8192, 4096, 4096, 8192
D, H, DH = 128, 4, 32
BN_SCALE = float(1.0 / np.sqrt(1.0 + 1e-5))
F32 = jnp.float32


def _gat_pre_body(x_ref, w_ref, asrc_ref, adst_ref, sel_ref, xw_ref, es_ref, edt_ref):
    xw = jnp.dot(x_ref[...], w_ref[...], preferred_element_type=F32)
    xw_ref[...] = xw
    sel = sel_ref[...]
    es_ref[...] = jnp.dot(xw * asrc_ref[...], sel, preferred_element_type=F32)
    td = xw * adst_ref[...]
    edt_ref[...] = jax.lax.dot_general(
        sel, td, (((0,), (1,)), ((), ())), preferred_element_type=F32)


def _gat_main_body(final):
    def body(a_ref, xw_ref, es_ref, edt_ref, s_ref, o_ref):
        a = a_ref[...]
        mask = a > 0
        xw = xw_ref[...]
        es = es_ref[...]
        edt = edt_ref[...]
        outs = []
        for h in range(H):
            z = es[:, h:h + 1] + edt[h:h + 1, :]
            e = jnp.where(z >= 0, z, 0.2 * z)
            e = jnp.where(mask, e, -1e9)
            m = jnp.max(e, axis=1, keepdims=True)
            p = jnp.exp(e - m)
            ssum = jnp.sum(p, axis=1, keepdims=True)
            alpha = p / ssum
            outs.append(jnp.dot(alpha, xw[:, h * DH:(h + 1) * DH],
                                preferred_element_type=F32))
        out = jnp.concatenate(outs, axis=1) * BN_SCALE
        if final:
            v = out + s_ref[...]
            out = jnp.where(v > 0, v, jnp.exp(jnp.minimum(v, 0.0)) - 1.0)
        o_ref[...] = out
    return body


def _gat_layer(X, A, W_flat, a_src_flat, a_dst_flat, sel, S0, final):
    xw, es, edt = pl.pallas_call(
        _gat_pre_body,
        out_shape=[jax.ShapeDtypeStruct((N_S, D), F32),
                   jax.ShapeDtypeStruct((N_S, H), F32),
                   jax.ShapeDtypeStruct((H, N_S), F32)],
    )(X, W_flat, a_src_flat, a_dst_flat, sel)

    BN = 256
    out = pl.pallas_call(
        _gat_main_body(final),
        grid=(N_S // BN,),
        in_specs=[
            pl.BlockSpec((BN, N_S), lambda i: (i, 0)),
            pl.BlockSpec((N_S, D), lambda i: (0, 0)),
            pl.BlockSpec((BN, H), lambda i: (i, 0)),
            pl.BlockSpec((H, N_S), lambda i: (0, 0)),
            pl.BlockSpec((BN, D), lambda i: (i, 0)),
        ],
        out_specs=pl.BlockSpec((BN, D), lambda i: (i, 0)),
        out_shape=jax.ShapeDtypeStruct((N_S, D), F32),
    )(A, xw, es, edt, S0)
    return out


def _bip_fwd_body(a_ref, x_ref, w_ref, b_ref, o_ref, acc_ref, rs_ref):
    k = pl.program_id(1)
    nk = pl.num_programs(1)
    a = a_ref[...]
    part = jnp.dot(a.astype(jnp.bfloat16), x_ref[...].astype(jnp.bfloat16),
                   preferred_element_type=F32)
    rpart = jnp.sum(a, axis=1, keepdims=True)

    @pl.when(k == 0)
    def _():
        acc_ref[...] = part
        rs_ref[...] = rpart

    @pl.when(k != 0)
    def _():
        acc_ref[...] += part
        rs_ref[...] += rpart

    @pl.when(k == nk - 1)
    def _():
        z = acc_ref[...] / jnp.clip(rs_ref[...], 1e-8, None)
        hh = jnp.dot(z, w_ref[...], preferred_element_type=F32) + b_ref[...]
        o_ref[...] = jnp.maximum(hh * BN_SCALE, 0.0)


def _bip_fwd(A, X, W, b, bt, bk):
    T, S = A.shape
    return pl.pallas_call(
        _bip_fwd_body,
        grid=(T // bt, S // bk),
        in_specs=[
            pl.BlockSpec((bt, bk), lambda i, k: (i, k)),
            pl.BlockSpec((bk, D), lambda i, k: (k, 0)),
            pl.BlockSpec((D, D), lambda i, k: (0, 0)),
            pl.BlockSpec((1, D), lambda i, k: (0, 0)),
        ],
        out_specs=pl.BlockSpec((bt, D), lambda i, k: (i, 0)),
        out_shape=jax.ShapeDtypeStruct((T, D), F32),
        scratch_shapes=[pltpu.VMEM((bt, D), F32), pltpu.VMEM((bt, 1), F32)],
    )(A, X, W, b)


def _bip_fused12_body(a_ref, x_ref, w0_ref, b0_ref, w1_ref, b1_ref,
                      o_ref, acc_ref, cs_ref):
    t = pl.program_id(0)
    nt = pl.num_programs(0)
    a = a_ref[...]
    a16 = a.astype(jnp.bfloat16)
    rs = jnp.clip(jnp.sum(a, axis=1, keepdims=True), 1e-8, None)
    z = jnp.dot(a16, x_ref[...].astype(jnp.bfloat16),
                preferred_element_type=F32) / rs
    y = jnp.maximum((jnp.dot(z, w0_ref[...], preferred_element_type=F32)
                     + b0_ref[...]) * BN_SCALE, 0.0)
    part2 = jax.lax.dot_general(
        a16, y.astype(jnp.bfloat16), (((0,), (0,)), ((), ())),
        preferred_element_type=F32)
    cpart = jnp.sum(a, axis=0, keepdims=True)

    @pl.when(t == 0)
    def _():
        acc_ref[...] = part2
        cs_ref[...] = cpart

    @pl.when(t != 0)
    def _():
        acc_ref[...] += part2
        cs_ref[...] += cpart

    @pl.when(t == nt - 1)
    def _():
        cs_col = jnp.transpose(cs_ref[...])
        z2 = acc_ref[...] / jnp.clip(cs_col, 1e-8, None)
        hh = jnp.dot(z2, w1_ref[...], preferred_element_type=F32) + b1_ref[...]
        o_ref[...] = jnp.maximum(hh * BN_SCALE, 0.0)


def _bip_fused12(A, X, W0, b0, W1, b1, bt):
    T, S = A.shape
    return pl.pallas_call(
        _bip_fused12_body,
        grid=(T // bt,),
        in_specs=[
            pl.BlockSpec((bt, S), lambda t: (t, 0)),
            pl.BlockSpec((S, D), lambda t: (0, 0)),
            pl.BlockSpec((D, D), lambda t: (0, 0)),
            pl.BlockSpec((1, D), lambda t: (0, 0)),
            pl.BlockSpec((D, D), lambda t: (0, 0)),
            pl.BlockSpec((1, D), lambda t: (0, 0)),
        ],
        out_specs=pl.BlockSpec((S, D), lambda t: (0, 0)),
        out_shape=jax.ShapeDtypeStruct((S, D), F32),
        scratch_shapes=[pltpu.VMEM((S, D), F32), pltpu.VMEM((1, S), F32)],
    )(A, X, W0, b0, W1, b1)


def _bip_bwd_body(a_ref, y_ref, w_ref, b_ref, o_ref, acc_ref, cs_ref):
    t = pl.program_id(1)
    nt = pl.num_programs(1)
    a = a_ref[...]
    part = jax.lax.dot_general(
        a, y_ref[...], (((0,), (0,)), ((), ())), preferred_element_type=F32)
    cpart = jnp.sum(a, axis=0, keepdims=True)

    @pl.when(t == 0)
    def _():
        acc_ref[...] = part
        cs_ref[...] = cpart

    @pl.when(t != 0)
    def _():
        acc_ref[...] += part
        cs_ref[...] += cpart

    @pl.when(t == nt - 1)
    def _():
        cs_col = jnp.transpose(cs_ref[...])
        z = acc_ref[...] / jnp.clip(cs_col, 1e-8, None)
        hh = jnp.dot(z, w_ref[...], preferred_element_type=F32) + b_ref[...]
        o_ref[...] = jnp.maximum(hh * BN_SCALE, 0.0)


def _bip_bwd(A, Y, W, b, bs, bkt):
    T, S = A.shape
    return pl.pallas_call(
        _bip_bwd_body,
        grid=(S // bs, T // bkt),
        in_specs=[
            pl.BlockSpec((bkt, bs), lambda i, t: (t, i)),
            pl.BlockSpec((bkt, D), lambda i, t: (t, 0)),
            pl.BlockSpec((D, D), lambda i, t: (0, 0)),
            pl.BlockSpec((1, D), lambda i, t: (0, 0)),
        ],
        out_specs=pl.BlockSpec((bs, D), lambda i, t: (i, 0)),
        out_shape=jax.ShapeDtypeStruct((S, D), F32),
        scratch_shapes=[pltpu.VMEM((bs, D), F32), pltpu.VMEM((1, bs), F32)],
    )(A, Y, W, b)


def _bip_stack(h_src, A, Ws, bs_, bt_f, bk_f, bt_12):
    h2 = _bip_fused12(A, h_src, Ws[0], bs_[0], Ws[1], bs_[1], bt_12)
    h3 = _bip_fwd(A, h2, Ws[2], bs_[2], bt_f, bk_f)
    return h3


def kernel(A_pre, A_qs, A_vs, A_rs, A_uq, S, Q, V, R, U,
           gat_W, gat_a_src, gat_a_dst,
           q_W, q_b, v_W, v_b, r_W, r_b, u_W, u_b):
    sel = jnp.repeat(jnp.eye(H, dtype=F32), DH, axis=0)

    x = S
    for i in range(3):
        x = _gat_layer(
            x, A_pre,
            gat_W[i].reshape(D, H * DH),
            gat_a_src[i].reshape(1, H * DH),
            gat_a_dst[i].reshape(1, H * DH),
            sel, S, final=(i == 2))
    h_s = x

    def _b(b):
        return [b[l].reshape(1, D) for l in range(3)]

    h_qa = _bip_stack(h_s, A_qs, q_W, _b(q_b), 512, 2048, 512)
    h_v = _bip_stack(h_s, A_vs, v_W, _b(v_b), 512, 2048, 512)
    h_r = _bip_stack(h_s, A_rs, r_W, _b(r_b), 512, 2048, 512)
    h_u = _bip_stack(h_qa, A_uq, u_W, _b(u_b), 512, 2048, 256)

    return jnp.concatenate([h_qa, h_v, h_r, h_u], axis=0)


# speedup vs baseline: 1.5213x; 1.0806x over previous
import jax
import jax.numpy as jnp
import numpy as np
from jax.experimental import pallas as pl
from jax.experimental.pallas import tpu as pltpu

N_S, N_Q, N_V, N_R, N_U = 2048, 8192, 4096, 4096, 8192
D, H, DH = 128, 4, 32
BN_SCALE = float(1.0 / np.sqrt(1.0 + 1e-5))
F32 = jnp.float32


def _gat_pre_body(x_ref, w_ref, asrc_ref, adst_ref, sel_ref,
                  xw_ref, u1_ref, u2_ref, v1_ref, v2_ref):
    xw = jnp.dot(x_ref[...], w_ref[...], preferred_element_type=F32)
    xw_ref[...] = xw
    sel = sel_ref[...]
    es = jnp.dot(xw * asrc_ref[...], sel, preferred_element_type=F32)
    edt = jax.lax.dot_general(
        sel, xw * adst_ref[...], (((0,), (1,)), ((), ())),
        preferred_element_type=F32)
    edmax = jnp.transpose(jnp.max(edt, axis=1, keepdims=True))
    zmax = es + edmax
    M = jnp.where(zmax >= 0, zmax, 0.2 * zmax)
    u1_ref[...] = jnp.exp(es - M)
    u2_ref[...] = jnp.exp(0.2 * es - M)
    v1_ref[...] = jnp.exp(edt)
    v2_ref[...] = jnp.exp(0.2 * edt)


def _gat_main_body(final):
    def body(a_ref, xw_ref, u1_ref, u2_ref, v1_ref, v2_ref, s_ref, o_ref):
        a = a_ref[...]
        xw = xw_ref[...].astype(jnp.bfloat16)
        u1 = u1_ref[...]
        u2 = u2_ref[...]
        v1 = v1_ref[...]
        v2 = v2_ref[...]
        outs = []
        for h in range(H):
            n1 = u1[:, h:h + 1] * v1[h:h + 1, :]
            n2 = u2[:, h:h + 1] * v2[h:h + 1, :]
            numer = jnp.where(a > 0, jnp.maximum(n1, n2), 0.0)
            denom = jnp.sum(numer, axis=1, keepdims=True)
            acc = jnp.dot(numer.astype(jnp.bfloat16), xw[:, h * DH:(h + 1) * DH],
                          preferred_element_type=F32)
            outs.append(acc / denom)
        out = jnp.concatenate(outs, axis=1) * BN_SCALE
        if final:
            v = out + s_ref[...]
            out = jnp.where(v > 0, v, jnp.exp(jnp.minimum(v, 0.0)) - 1.0)
        o_ref[...] = out
    return body


def _gat_layer(X, A, W_flat, a_src_flat, a_dst_flat, sel, S0, final):
    xw, u1, u2, v1, v2 = pl.pallas_call(
        _gat_pre_body,
        out_shape=[jax.ShapeDtypeStruct((N_S, D), F32),
                   jax.ShapeDtypeStruct((N_S, H), F32),
                   jax.ShapeDtypeStruct((N_S, H), F32),
                   jax.ShapeDtypeStruct((H, N_S), F32),
                   jax.ShapeDtypeStruct((H, N_S), F32)],
    )(X, W_flat, a_src_flat, a_dst_flat, sel)

    BN = 256
    out = pl.pallas_call(
        _gat_main_body(final),
        grid=(N_S // BN,),
        in_specs=[
            pl.BlockSpec((BN, N_S), lambda i: (i, 0)),
            pl.BlockSpec((N_S, D), lambda i: (0, 0)),
            pl.BlockSpec((BN, H), lambda i: (i, 0)),
            pl.BlockSpec((BN, H), lambda i: (i, 0)),
            pl.BlockSpec((H, N_S), lambda i: (0, 0)),
            pl.BlockSpec((H, N_S), lambda i: (0, 0)),
            pl.BlockSpec((BN, D), lambda i: (i, 0)),
        ],
        out_specs=pl.BlockSpec((BN, D), lambda i: (i, 0)),
        out_shape=jax.ShapeDtypeStruct((N_S, D), F32),
    )(A, xw, u1, u2, v1, v2, S0)
    return out


def _bip_fwd_body(a_ref, x_ref, w_ref, b_ref, o_ref, acc_ref, rs_ref):
    k = pl.program_id(1)
    nk = pl.num_programs(1)
    a = a_ref[...]
    part = jnp.dot(a.astype(jnp.bfloat16), x_ref[...].astype(jnp.bfloat16),
                   preferred_element_type=F32)
    rpart = jnp.sum(a, axis=1, keepdims=True)

    @pl.when(k == 0)
    def _():
        acc_ref[...] = part
        rs_ref[...] = rpart

    @pl.when(k != 0)
    def _():
        acc_ref[...] += part
        rs_ref[...] += rpart

    @pl.when(k == nk - 1)
    def _():
        z = acc_ref[...] / jnp.clip(rs_ref[...], 1e-8, None)
        hh = jnp.dot(z, w_ref[...], preferred_element_type=F32) + b_ref[...]
        o_ref[...] = jnp.maximum(hh * BN_SCALE, 0.0)


def _bip_fwd(A, X, W, b, bt, bk):
    T, S = A.shape
    return pl.pallas_call(
        _bip_fwd_body,
        grid=(T // bt, S // bk),
        in_specs=[
            pl.BlockSpec((bt, bk), lambda i, k: (i, k)),
            pl.BlockSpec((bk, D), lambda i, k: (k, 0)),
            pl.BlockSpec((D, D), lambda i, k: (0, 0)),
            pl.BlockSpec((1, D), lambda i, k: (0, 0)),
        ],
        out_specs=pl.BlockSpec((bt, D), lambda i, k: (i, 0)),
        out_shape=jax.ShapeDtypeStruct((T, D), F32),
        scratch_shapes=[pltpu.VMEM((bt, D), F32), pltpu.VMEM((bt, 1), F32)],
    )(A, X, W, b)


def _bip_fused12_body(a_ref, x_ref, w0_ref, b0_ref, w1_ref, b1_ref,
                      o_ref, acc_ref, cs_ref):
    t = pl.program_id(0)
    nt = pl.num_programs(0)
    a = a_ref[...]
    a16 = a.astype(jnp.bfloat16)
    rs = jnp.clip(jnp.sum(a, axis=1, keepdims=True), 1e-8, None)
    z = jnp.dot(a16, x_ref[...].astype(jnp.bfloat16),
                preferred_element_type=F32) / rs
    y = jnp.maximum((jnp.dot(z, w0_ref[...], preferred_element_type=F32)
                     + b0_ref[...]) * BN_SCALE, 0.0)
    part2 = jax.lax.dot_general(
        a16, y.astype(jnp.bfloat16), (((0,), (0,)), ((), ())),
        preferred_element_type=F32)
    cpart = jnp.sum(a, axis=0, keepdims=True)

    @pl.when(t == 0)
    def _():
        acc_ref[...] = part2
        cs_ref[...] = cpart

    @pl.when(t != 0)
    def _():
        acc_ref[...] += part2
        cs_ref[...] += cpart

    @pl.when(t == nt - 1)
    def _():
        cs_col = jnp.transpose(cs_ref[...])
        z2 = acc_ref[...] / jnp.clip(cs_col, 1e-8, None)
        hh = jnp.dot(z2, w1_ref[...], preferred_element_type=F32) + b1_ref[...]
        o_ref[...] = jnp.maximum(hh * BN_SCALE, 0.0)


def _bip_fused12(A, X, W0, b0, W1, b1, bt):
    T, S = A.shape
    return pl.pallas_call(
        _bip_fused12_body,
        grid=(T // bt,),
        in_specs=[
            pl.BlockSpec((bt, S), lambda t: (t, 0)),
            pl.BlockSpec((S, D), lambda t: (0, 0)),
            pl.BlockSpec((D, D), lambda t: (0, 0)),
            pl.BlockSpec((1, D), lambda t: (0, 0)),
            pl.BlockSpec((D, D), lambda t: (0, 0)),
            pl.BlockSpec((1, D), lambda t: (0, 0)),
        ],
        out_specs=pl.BlockSpec((S, D), lambda t: (0, 0)),
        out_shape=jax.ShapeDtypeStruct((S, D), F32),
        scratch_shapes=[pltpu.VMEM((S, D), F32), pltpu.VMEM((1, S), F32)],
    )(A, X, W0, b0, W1, b1)


def _bip_bwd_body(a_ref, y_ref, w_ref, b_ref, o_ref, acc_ref, cs_ref):
    t = pl.program_id(1)
    nt = pl.num_programs(1)
    a = a_ref[...]
    part = jax.lax.dot_general(
        a, y_ref[...], (((0,), (0,)), ((), ())), preferred_element_type=F32)
    cpart = jnp.sum(a, axis=0, keepdims=True)

    @pl.when(t == 0)
    def _():
        acc_ref[...] = part
        cs_ref[...] = cpart

    @pl.when(t != 0)
    def _():
        acc_ref[...] += part
        cs_ref[...] += cpart

    @pl.when(t == nt - 1)
    def _():
        cs_col = jnp.transpose(cs_ref[...])
        z = acc_ref[...] / jnp.clip(cs_col, 1e-8, None)
        hh = jnp.dot(z, w_ref[...], preferred_element_type=F32) + b_ref[...]
        o_ref[...] = jnp.maximum(hh * BN_SCALE, 0.0)


def _bip_bwd(A, Y, W, b, bs, bkt):
    T, S = A.shape
    return pl.pallas_call(
        _bip_bwd_body,
        grid=(S // bs, T // bkt),
        in_specs=[
            pl.BlockSpec((bkt, bs), lambda i, t: (t, i)),
            pl.BlockSpec((bkt, D), lambda i, t: (t, 0)),
            pl.BlockSpec((D, D), lambda i, t: (0, 0)),
            pl.BlockSpec((1, D), lambda i, t: (0, 0)),
        ],
        out_specs=pl.BlockSpec((bs, D), lambda i, t: (i, 0)),
        out_shape=jax.ShapeDtypeStruct((S, D), F32),
        scratch_shapes=[pltpu.VMEM((bs, D), F32), pltpu.VMEM((1, bs), F32)],
    )(A, Y, W, b)


def _bip_stack(h_src, A, Ws, bs_, bt_f, bk_f, bt_12):
    h2 = _bip_fused12(A, h_src, Ws[0], bs_[0], Ws[1], bs_[1], bt_12)
    h3 = _bip_fwd(A, h2, Ws[2], bs_[2], bt_f, bk_f)
    return h3


def kernel(A_pre, A_qs, A_vs, A_rs, A_uq, S, Q, V, R, U,
           gat_W, gat_a_src, gat_a_dst,
           q_W, q_b, v_W, v_b, r_W, r_b, u_W, u_b):
    sel = jnp.repeat(jnp.eye(H, dtype=F32), DH, axis=0)

    x = S
    for i in range(3):
        x = _gat_layer(
            x, A_pre,
            gat_W[i].reshape(D, H * DH),
            gat_a_src[i].reshape(1, H * DH),
            gat_a_dst[i].reshape(1, H * DH),
            sel, S, final=(i == 2))
    h_s = x

    def _b(b):
        return [b[l].reshape(1, D) for l in range(3)]

    h_qa = _bip_stack(h_s, A_qs, q_W, _b(q_b), 512, 2048, 512)
    h_v = _bip_stack(h_s, A_vs, v_W, _b(v_b), 512, 2048, 512)
    h_r = _bip_stack(h_s, A_rs, r_W, _b(r_b), 512, 2048, 512)
    h_u = _bip_stack(h_qa, A_uq, u_W, _b(u_b), 512, 2048, 256)

    return jnp.concatenate([h_qa, h_v, h_r, h_u], axis=0)


# speedup vs baseline: 1.5308x; 1.0062x over previous
import jax
import jax.numpy as jnp
import numpy as np
from jax.experimental import pallas as pl
from jax.experimental.pallas import tpu as pltpu

N_S, N_Q, N_V, N_R, N_U = 2048, 8192, 4096, 4096, 8192
D, H, DH = 128, 4, 32
BN_SCALE = float(1.0 / np.sqrt(1.0 + 1e-5))
F32 = jnp.float32


def _gat_pre_body(x_ref, w_ref, asrc_ref, adst_ref, sel_ref,
                  xw_ref, u1_ref, u2_ref, v1_ref, v2_ref):
    xw = jnp.dot(x_ref[...], w_ref[...], preferred_element_type=F32)
    xw_ref[...] = xw
    sel = sel_ref[...]
    es = jnp.dot(xw * asrc_ref[...], sel, preferred_element_type=F32)
    edt = jax.lax.dot_general(
        sel, xw * adst_ref[...], (((0,), (1,)), ((), ())),
        preferred_element_type=F32)
    edmax = jnp.transpose(jnp.max(edt, axis=1, keepdims=True))
    zmax = es + edmax
    M = jnp.where(zmax >= 0, zmax, 0.2 * zmax)
    u1_ref[...] = jnp.exp(es - M)
    u2_ref[...] = jnp.exp(0.2 * es - M)
    v1_ref[...] = jnp.exp(edt)
    v2_ref[...] = jnp.exp(0.2 * edt)


def _gat_main_body(final):
    def body(a_ref, xw_ref, u1_ref, u2_ref, v1_ref, v2_ref, s_ref, o_ref):
        a = a_ref[...]
        xw = xw_ref[...].astype(jnp.bfloat16)
        u1 = u1_ref[...]
        u2 = u2_ref[...]
        v1 = v1_ref[...]
        v2 = v2_ref[...]
        outs = []
        for h in range(H):
            n1 = u1[:, h:h + 1] * v1[h:h + 1, :]
            n2 = u2[:, h:h + 1] * v2[h:h + 1, :]
            numer = jnp.where(a > 0, jnp.maximum(n1, n2), 0.0)
            denom = jnp.sum(numer, axis=1, keepdims=True)
            acc = jnp.dot(numer.astype(jnp.bfloat16), xw[:, h * DH:(h + 1) * DH],
                          preferred_element_type=F32)
            outs.append(acc / denom)
        out = jnp.concatenate(outs, axis=1) * BN_SCALE
        if final:
            v = out + s_ref[...]
            out = jnp.where(v > 0, v, jnp.exp(jnp.minimum(v, 0.0)) - 1.0)
        o_ref[...] = out
    return body


def _gat_layer(X, A, W_flat, a_src_flat, a_dst_flat, sel, S0, final):
    xw, u1, u2, v1, v2 = pl.pallas_call(
        _gat_pre_body,
        out_shape=[jax.ShapeDtypeStruct((N_S, D), F32),
                   jax.ShapeDtypeStruct((N_S, H), F32),
                   jax.ShapeDtypeStruct((N_S, H), F32),
                   jax.ShapeDtypeStruct((H, N_S), F32),
                   jax.ShapeDtypeStruct((H, N_S), F32)],
    )(X, W_flat, a_src_flat, a_dst_flat, sel)

    BN = 256
    out = pl.pallas_call(
        _gat_main_body(final),
        grid=(N_S // BN,),
        in_specs=[
            pl.BlockSpec((BN, N_S), lambda i: (i, 0)),
            pl.BlockSpec((N_S, D), lambda i: (0, 0)),
            pl.BlockSpec((BN, H), lambda i: (i, 0)),
            pl.BlockSpec((BN, H), lambda i: (i, 0)),
            pl.BlockSpec((H, N_S), lambda i: (0, 0)),
            pl.BlockSpec((H, N_S), lambda i: (0, 0)),
            pl.BlockSpec((BN, D), lambda i: (i, 0)),
        ],
        out_specs=pl.BlockSpec((BN, D), lambda i: (i, 0)),
        out_shape=jax.ShapeDtypeStruct((N_S, D), F32),
    )(A, xw, u1, u2, v1, v2, S0)
    return out


def _bip_fwd_body(a_ref, inv_ref, x_ref, w_ref, b_ref, o_ref, acc_ref):
    k = pl.program_id(1)
    nk = pl.num_programs(1)
    part = jnp.dot(a_ref[...], x_ref[...].astype(jnp.bfloat16),
                   preferred_element_type=F32)

    @pl.when(k == 0)
    def _():
        acc_ref[...] = part

    @pl.when(k != 0)
    def _():
        acc_ref[...] += part

    @pl.when(k == nk - 1)
    def _():
        z = acc_ref[...] * inv_ref[...]
        hh = jnp.dot(z, w_ref[...], preferred_element_type=F32) + b_ref[...]
        o_ref[...] = jnp.maximum(hh * BN_SCALE, 0.0)


def _bip_fwd(A16, inv, X, W, b, bt, bk):
    T, S = A16.shape
    return pl.pallas_call(
        _bip_fwd_body,
        grid=(T // bt, S // bk),
        in_specs=[
            pl.BlockSpec((bt, bk), lambda i, k: (i, k)),
            pl.BlockSpec((bt, 1), lambda i, k: (i, 0)),
            pl.BlockSpec((bk, D), lambda i, k: (k, 0)),
            pl.BlockSpec((D, D), lambda i, k: (0, 0)),
            pl.BlockSpec((1, D), lambda i, k: (0, 0)),
        ],
        out_specs=pl.BlockSpec((bt, D), lambda i, k: (i, 0)),
        out_shape=jax.ShapeDtypeStruct((T, D), F32),
        scratch_shapes=[pltpu.VMEM((bt, D), F32)],
    )(A16, inv, X, W, b)


def _bip_fused12_body(a_ref, x_ref, w0_ref, b0_ref, w1_ref, b1_ref,
                      o_ref, a16_ref, inv_ref, acc_ref, cs_ref):
    t = pl.program_id(0)
    nt = pl.num_programs(0)
    a = a_ref[...]
    a16 = a.astype(jnp.bfloat16)
    a16_ref[...] = a16
    inv = 1.0 / jnp.clip(jnp.sum(a, axis=1, keepdims=True), 1e-8, None)
    inv_ref[...] = inv
    z = jnp.dot(a16, x_ref[...].astype(jnp.bfloat16),
                preferred_element_type=F32) * inv
    y = jnp.maximum((jnp.dot(z, w0_ref[...], preferred_element_type=F32)
                     + b0_ref[...]) * BN_SCALE, 0.0)
    part2 = jax.lax.dot_general(
        a16, y.astype(jnp.bfloat16), (((0,), (0,)), ((), ())),
        preferred_element_type=F32)
    cpart = jnp.sum(a, axis=0, keepdims=True)

    @pl.when(t == 0)
    def _():
        acc_ref[...] = part2
        cs_ref[...] = cpart

    @pl.when(t != 0)
    def _():
        acc_ref[...] += part2
        cs_ref[...] += cpart

    @pl.when(t == nt - 1)
    def _():
        cs_col = jnp.transpose(cs_ref[...])
        z2 = acc_ref[...] / jnp.clip(cs_col, 1e-8, None)
        hh = jnp.dot(z2, w1_ref[...], preferred_element_type=F32) + b1_ref[...]
        o_ref[...] = jnp.maximum(hh * BN_SCALE, 0.0)


def _bip_fused12(A, X, W0, b0, W1, b1, bt):
    T, S = A.shape
    return pl.pallas_call(
        _bip_fused12_body,
        grid=(T // bt,),
        in_specs=[
            pl.BlockSpec((bt, S), lambda t: (t, 0)),
            pl.BlockSpec((S, D), lambda t: (0, 0)),
            pl.BlockSpec((D, D), lambda t: (0, 0)),
            pl.BlockSpec((1, D), lambda t: (0, 0)),
            pl.BlockSpec((D, D), lambda t: (0, 0)),
            pl.BlockSpec((1, D), lambda t: (0, 0)),
        ],
        out_specs=[
            pl.BlockSpec((S, D), lambda t: (0, 0)),
            pl.BlockSpec((bt, S), lambda t: (t, 0)),
            pl.BlockSpec((bt, 1), lambda t: (t, 0)),
        ],
        out_shape=[jax.ShapeDtypeStruct((S, D), F32),
                   jax.ShapeDtypeStruct((T, S), jnp.bfloat16),
                   jax.ShapeDtypeStruct((T, 1), F32)],
        scratch_shapes=[pltpu.VMEM((S, D), F32), pltpu.VMEM((1, S), F32)],
    )(A, X, W0, b0, W1, b1)


def _bip_stack(h_src, A, Ws, bs_, bt_f, bk_f, bt_12):
    h2, a16, inv = _bip_fused12(A, h_src, Ws[0], bs_[0], Ws[1], bs_[1], bt_12)
    h3 = _bip_fwd(a16, inv, h2, Ws[2], bs_[2], bt_f, bk_f)
    return h3


def kernel(A_pre, A_qs, A_vs, A_rs, A_uq, S, Q, V, R, U,
           gat_W, gat_a_src, gat_a_dst,
           q_W, q_b, v_W, v_b, r_W, r_b, u_W, u_b):
    sel = jnp.repeat(jnp.eye(H, dtype=F32), DH, axis=0)

    x = S
    for i in range(3):
        x = _gat_layer(
            x, A_pre,
            gat_W[i].reshape(D, H * DH),
            gat_a_src[i].reshape(1, H * DH),
            gat_a_dst[i].reshape(1, H * DH),
            sel, S, final=(i == 2))
    h_s = x

    def _b(b):
        return [b[l].reshape(1, D) for l in range(3)]

    h_qa = _bip_stack(h_s, A_qs, q_W, _b(q_b), 512, 2048, 512)
    h_v = _bip_stack(h_s, A_vs, v_W, _b(v_b), 512, 2048, 512)
    h_r = _bip_stack(h_s, A_rs, r_W, _b(r_b), 512, 2048, 512)
    h_u = _bip_stack(h_qa, A_uq, u_W, _b(u_b), 512, 2048, 256)

    return jnp.concatenate([h_qa, h_v, h_r, h_u], axis=0)
